# baseline (speedup 1.0000x reference)
import jax
import jax.numpy as jnp
from jax import lax
from jax.experimental import pallas as pl
from jax.experimental.pallas import tpu as pltpu

B = 8
N = 4096
D = 2048
K = 10
T = 256
NT = N // T


def _body(x_ref, wa_ref, ba_ref, wc_ref, bc_ref, pred_ref, scores_ref, acc_ref):
    t = pl.program_id(0)
    xt = x_ref[...].reshape(B * T, D)
    s = lax.dot_general(
        xt, wa_ref[...],
        (((1,), (0,)), ((), ())),
        precision=lax.Precision.HIGHEST,
        preferred_element_type=jnp.float32,
    )
    s2 = s.reshape(B, T) + ba_ref[0, 0]
    scores_ref[...] = s2
    acc_ref[:, pl.ds(t * T, T)] = s2

    @pl.when(t == NT - 1)
    def _finish():
        s_all = acc_ref[...]
        ii = lax.broadcasted_iota(jnp.int32, (B, N), 1)
        pred = jnp.zeros((B, 1), jnp.float32)

        st = s_all
        for k in range(K):
            m = jnp.max(st, axis=1, keepdims=True)
            pred += m * wc_ref[0, k]
            first = jnp.min(jnp.where(st == m, ii, jnp.int32(N)), axis=1,
                            keepdims=True)
            st = jnp.where(ii == first, -jnp.inf, st)

        sb = s_all
        for k in range(K):
            m = jnp.min(sb, axis=1, keepdims=True)
            pred += m * wc_ref[0, K + k]
            first = jnp.min(jnp.where(sb == m, ii, jnp.int32(N)), axis=1,
                            keepdims=True)
            sb = jnp.where(ii == first, jnp.inf, sb)

        pred_ref[...] = pred + bc_ref[0, 0]


def kernel(x, W_attn, b_attn, W_cls, b_cls):
    wa = W_attn.reshape(D, 1)
    ba = b_attn.reshape(1, 1)
    wc = W_cls.reshape(1, 2 * K)
    bc = b_cls.reshape(1, 1)

    pred, scores = pl.pallas_call(
        _body,
        grid=(NT,),
        in_specs=[
            pl.BlockSpec((B, T, D), lambda t: (0, t, 0)),
            pl.BlockSpec((D, 1), lambda t: (0, 0)),
            pl.BlockSpec((1, 1), lambda t: (0, 0)),
            pl.BlockSpec((1, 2 * K), lambda t: (0, 0)),
            pl.BlockSpec((1, 1), lambda t: (0, 0)),
        ],
        out_specs=[
            pl.BlockSpec((B, 1), lambda t: (0, 0)),
            pl.BlockSpec((B, T), lambda t: (0, t)),
        ],
        out_shape=[
            jax.ShapeDtypeStruct((B, 1), jnp.float32),
            jax.ShapeDtypeStruct((B, N), jnp.float32),
        ],
        scratch_shapes=[pltpu.VMEM((B, N), jnp.float32)],
    )(x, wa, ba, wc, bc)
    return (pred, scores)


# speedup vs baseline: 2.6428x; 2.6428x over previous
import jax
import jax.numpy as jnp
from jax import lax
from jax.experimental import pallas as pl
from jax.experimental.pallas import tpu as pltpu

B = 8
N = 4096
D = 2048
K = 10
T = 256
NT = N // T


def _body(x_ref, wa_ref, ba_ref, wc_ref, bc_ref, pred_ref, scores_ref, acc_ref):
    t = pl.program_id(0)
    xt = x_ref[...]
    w = wa_ref[...]
    s2 = jnp.sum(xt * w, axis=2) + ba_ref[0, 0]
    scores_ref[...] = s2
    acc_ref[:, pl.ds(t * T, T)] = s2

    @pl.when(t == NT - 1)
    def _finish():
        s_all = acc_ref[...]
        ii = lax.broadcasted_iota(jnp.int32, (B, N), 1)
        pred = jnp.zeros((B, 1), jnp.float32)

        st = s_all
        for k in range(K):
            m = jnp.max(st, axis=1, keepdims=True)
            pred += m * wc_ref[0, k]
            first = jnp.min(jnp.where(st == m, ii, jnp.int32(N)), axis=1,
                            keepdims=True)
            st = jnp.where(ii == first, -jnp.inf, st)

        sb = s_all
        for k in range(K):
            m = jnp.min(sb, axis=1, keepdims=True)
            pred += m * wc_ref[0, K + k]
            first = jnp.min(jnp.where(sb == m, ii, jnp.int32(N)), axis=1,
                            keepdims=True)
            sb = jnp.where(ii == first, jnp.inf, sb)

        pred_ref[...] = pred + bc_ref[0, 0]


def kernel(x, W_attn, b_attn, W_cls, b_cls):
    wa = W_attn.reshape(1, 1, D)
    ba = b_attn.reshape(1, 1)
    wc = W_cls.reshape(1, 2 * K)
    bc = b_cls.reshape(1, 1)

    pred, scores = pl.pallas_call(
        _body,
        grid=(NT,),
        in_specs=[
            pl.BlockSpec((B, T, D), lambda t: (0, t, 0)),
            pl.BlockSpec((1, 1, D), lambda t: (0, 0, 0)),
            pl.BlockSpec((1, 1), lambda t: (0, 0)),
            pl.BlockSpec((1, 2 * K), lambda t: (0, 0)),
            pl.BlockSpec((1, 1), lambda t: (0, 0)),
        ],
        out_specs=[
            pl.BlockSpec((B, 1), lambda t: (0, 0)),
            pl.BlockSpec((B, T), lambda t: (0, t)),
        ],
        out_shape=[
            jax.ShapeDtypeStruct((B, 1), jnp.float32),
            jax.ShapeDtypeStruct((B, N), jnp.float32),
        ],
        scratch_shapes=[pltpu.VMEM((B, N), jnp.float32)],
    )(x, wa, ba, wc, bc)
    return (pred, scores)
